# baseline (speedup 1.0000x reference)
import jax
import jax.numpy as jnp
from jax import lax
from jax.experimental import pallas as pl
from jax.experimental.pallas import tpu as pltpu
from jax.experimental.pallas import tpu_sc as plsc

NUM_CORES = 2
NUM_SUBCORES = 16
NUM_WORKERS = NUM_CORES * NUM_SUBCORES
LANES = 16
BLK = 128
CAP = 768
CODES_CAP = 784


def _extract_body(sc0_hbm, sc1_hbm, bst0_hbm, bst1_hbm, pp0_hbm, pp1_hbm,
                  w0t_hbm, w1t_hbm, e0_hbm, e1_hbm,
                  tile_a, tile_b, rows_v, idx_v, codes_stage, bst_stage,
                  sem_a, sem_b, sem_sc):
    dim = w0t_hbm.shape[0]
    vocab = w0t_hbm.shape[1]
    n_blocks = (vocab + BLK - 1) // BLK
    bpw = (n_blocks + NUM_WORKERS - 1) // NUM_WORKERS
    max_vb = n_blocks - 1
    wid = lax.axis_index("s") * NUM_CORES + lax.axis_index("c")
    d16 = lax.iota(jnp.int32, LANES)

    for (wt_hbm, sc_hbm, bst_hbm, pp_hbm, e_hbm) in (
            (w0t_hbm, sc0_hbm, bst0_hbm, pp0_hbm, e0_hbm),
            (w1t_hbm, sc1_hbm, bst1_hbm, pp1_hbm, e1_hbm)):
        pltpu.sync_copy(bst_hbm.at[wid], bst_stage)
        head = bst_stage[pl.ds(0, LANES)]
        seg0 = head[0]
        wend = bst_stage[pl.ds(bpw - 8, LANES)][8]
        seg0a = pl.multiple_of((seg0 >> 3) << 3, 8)
        pltpu.sync_copy(sc_hbm.at[pl.ds(seg0a, CODES_CAP)], codes_stage)

        def vb_of(j):
            return jnp.minimum(wid * bpw + j, max_vb)

        def issue(j, tile, sem):
            return pltpu.async_copy(
                wt_hbm.at[:, pl.ds(vb_of(j) * BLK, BLK)], tile, sem)

        issue(0, tile_a, sem_a)
        issue(1, tile_b, sem_b)

        def pair_body(i, carry):
            for bsel, (tile, sem) in enumerate(((tile_a, sem_a),
                                                (tile_b, sem_b))):
                j = 2 * i + bsel
                pltpu.make_async_copy(
                    wt_hbm.at[:, pl.ds(0, BLK)], tile, sem).wait()
                bvec = bst_stage[pl.ds(j, LANES)]
                s_b = jnp.minimum(bvec[0], wend)
                e_b = jnp.minimum(bvec[1], wend)

                def elem_body(k, c2):
                    k_off = jnp.minimum(k - seg0a, CODES_CAP - LANES)
                    c = codes_stage[pl.ds(k_off, LANES)][0]
                    lane = c & (BLK - 1)
                    pos = jnp.minimum(k - seg0, CAP - 1)
                    cols = jnp.full((LANES,), lane, jnp.int32)
                    for m in range(dim // LANES):
                        col = plsc.load_gather(
                            tile, [m * LANES + d16, cols])
                        rows_v[pos, pl.ds(m * LANES, LANES)] = col
                    return c2

                lax.fori_loop(s_b, e_b, elem_body, 0)
                issue(j + 2, tile, sem)
            return carry

        lax.fori_loop(0, (bpw + 1) // 2, pair_body, 0)
        pltpu.make_async_copy(wt_hbm.at[:, pl.ds(0, BLK)], tile_a, sem_a).wait()
        pltpu.make_async_copy(wt_hbm.at[:, pl.ds(0, BLK)], tile_b, sem_b).wait()

        pltpu.sync_copy(pp_hbm.at[wid], idx_v)
        pltpu.async_copy(rows_v, e_hbm.at[idx_v], sem_sc).wait()


def _dot_body(e0_hbm, e1_hbm, c0_hbm, c1_hbm, b0_hbm, b1_hbm, out_hbm,
              e0_v, e1_v, c0_v, c1_v, bb0_v, bb1_v, out_v, sem):
    b_per_w = out_v.shape[0]
    chunk = e0_v.shape[0]
    dim = e0_v.shape[1] // 2
    wid = lax.axis_index("s") * NUM_CORES + lax.axis_index("c")
    base = wid * b_per_w
    d16 = lax.iota(jnp.int32, LANES)

    for c in range(b_per_w // chunk):
        cbase = base + c * chunk
        pltpu.sync_copy(c0_hbm.at[pl.ds(cbase, chunk)], c0_v)
        pltpu.sync_copy(c1_hbm.at[pl.ds(cbase, chunk)], c1_v)
        copies = [
            pltpu.async_copy(e0_hbm.at[pl.ds(cbase, chunk)], e0_v, sem),
            pltpu.async_copy(e1_hbm.at[pl.ds(cbase, chunk)], e1_v, sem),
            pltpu.async_copy(b0_hbm.at[c0_v], bb0_v, sem),
            pltpu.async_copy(b1_hbm.at[c1_v], bb1_v, sem),
        ]
        for cp in copies:
            cp.wait()

        def grp_body(g, carry):
            s = g * LANES
            rows = s + d16
            acc = bb0_v[pl.ds(s, LANES)] + bb1_v[pl.ds(s, LANES)]
            for d in range(dim):
                cols = jnp.full((LANES,), d, jnp.int32)
                acc = acc + plsc.load_gather(e0_v, [rows, cols]) * \
                    plsc.load_gather(e1_v, [rows, cols])
            out_v[pl.ds(c * chunk + s, LANES)] = acc
            return carry

        lax.fori_loop(0, chunk // LANES, grp_body, 0)

    pltpu.sync_copy(out_v, out_hbm.at[pl.ds(base, b_per_w)])


def kernel(x, W0, W1, b0, b1):
    batch = x.shape[0]
    vocab, dim = W0.shape
    n_blocks = (vocab + BLK - 1) // BLK
    bpw = (n_blocks + NUM_WORKERS - 1) // NUM_WORKERS
    b_per_w = batch // NUM_WORKERS

    codes0 = x[:, 0].astype(jnp.int32)
    codes1 = x[:, 1].astype(jnp.int32)

    def field_setup(codes):
        perm = jnp.argsort(codes).astype(jnp.int32)
        sc = codes[perm]
        qblocks = (jnp.arange(NUM_WORKERS * bpw + 280, dtype=jnp.int32)
                   * BLK).astype(jnp.int32)
        bs = jnp.searchsorted(sc, qblocks).astype(jnp.int32)
        rowidx = (jnp.arange(NUM_WORKERS)[:, None] * bpw
                  + jnp.arange(272)[None, :])
        bstpad = bs[rowidx]
        wstart = bs[jnp.arange(NUM_WORKERS) * bpw]
        wend = bs[jnp.arange(NUM_WORKERS) * bpw + bpw]
        posg = wstart[:, None] + jnp.arange(CAP)[None, :]
        valid = posg < wend[:, None]
        permpad = jnp.where(valid, perm[jnp.minimum(posg, batch - 1)],
                            batch).astype(jnp.int32)
        scp = jnp.pad(sc, (0, CODES_CAP))
        return scp, bstpad, permpad

    sc0p, bst0, pp0 = field_setup(codes0)
    sc1p, bst1, pp1 = field_setup(codes1)

    mesh = plsc.VectorSubcoreMesh(core_axis_name="c", subcore_axis_name="s")
    params = pltpu.CompilerParams(
        needs_layout_passes=False, use_tc_tiling_on_sc=True)

    extract = pl.kernel(
        _extract_body,
        out_type=(jax.ShapeDtypeStruct((batch + 1, 2 * dim), jnp.float32),
                  jax.ShapeDtypeStruct((batch + 1, 2 * dim), jnp.float32)),
        mesh=mesh,
        compiler_params=params,
        scratch_types=[
            pltpu.VMEM((dim, BLK), jnp.float32),
            pltpu.VMEM((dim, BLK), jnp.float32),
            pltpu.VMEM((CAP, 2 * dim), jnp.float32),
            pltpu.VMEM((CAP,), jnp.int32),
            pltpu.VMEM((CODES_CAP,), jnp.int32),
            pltpu.VMEM((272,), jnp.int32),
            pltpu.SemaphoreType.DMA,
            pltpu.SemaphoreType.DMA,
            pltpu.SemaphoreType.DMA,
        ],
    )
    e0, e1 = extract(sc0p, sc1p, bst0, bst1, pp0, pp1, W0.T, W1.T)

    dot = pl.kernel(
        _dot_body,
        out_type=jax.ShapeDtypeStruct((batch,), jnp.float32),
        mesh=mesh,
        compiler_params=params,
        scratch_types=[
            pltpu.VMEM((b_per_w // 2, 2 * dim), jnp.float32),
            pltpu.VMEM((b_per_w // 2, 2 * dim), jnp.float32),
            pltpu.VMEM((b_per_w // 2,), jnp.int32),
            pltpu.VMEM((b_per_w // 2,), jnp.int32),
            pltpu.VMEM((b_per_w // 2,), jnp.float32),
            pltpu.VMEM((b_per_w // 2,), jnp.float32),
            pltpu.VMEM((b_per_w,), jnp.float32),
            pltpu.SemaphoreType.DMA,
        ],
    )
    return dot(e0, e1, codes0, codes1, b0.reshape(-1), b1.reshape(-1))


# speedup vs baseline: 1.7636x; 1.7636x over previous
import jax
import jax.numpy as jnp
from jax import lax
from jax.experimental import pallas as pl
from jax.experimental.pallas import tpu as pltpu
from jax.experimental.pallas import tpu_sc as plsc

NUM_CORES = 2
NUM_SUBCORES = 16
NUM_WORKERS = NUM_CORES * NUM_SUBCORES
LANES = 16
BLK = 128
CAP = 784
NBUF = 3
DSPLIT = 8


def _extract_body(sc0_hbm, sc1_hbm, ws0_hbm, ws1_hbm, pm0_hbm, pm1_hbm,
                  w0t_hbm, w1t_hbm, e0_hbm, e1_hbm,
                  tile0, tile1, tile2, rows_v, codes_v, perm_v, wst_v,
                  sem0, sem1, sem2, sem_sc):
    dim = w0t_hbm.shape[0]
    vocab = w0t_hbm.shape[1]
    batch = e0_hbm.shape[0] - 1
    n_blocks = (vocab + BLK - 1) // BLK
    bpw = (n_blocks + NUM_WORKERS - 1) // NUM_WORKERS
    max_vb = n_blocks - 1
    n_slots = NBUF * ((bpw + NBUF - 1) // NBUF)
    wid = lax.axis_index("s") * NUM_CORES + lax.axis_index("c")
    d16 = lax.iota(jnp.int32, LANES)
    tiles = (tile0, tile1, tile2)
    sems = (sem0, sem1, sem2)

    for (wt_hbm, sc_hbm, ws_hbm, pm_hbm, e_hbm) in (
            (w0t_hbm, sc0_hbm, ws0_hbm, pm0_hbm, e0_hbm),
            (w1t_hbm, sc1_hbm, ws1_hbm, pm1_hbm, e1_hbm)):
        pltpu.sync_copy(ws_hbm, wst_v)
        wvec = wst_v[pl.ds(wid, LANES)]
        seg0 = wvec[0]
        wend = wvec[1]
        seg0a = pl.multiple_of((seg0 >> 3) << 3, 8)
        pltpu.sync_copy(sc_hbm.at[pl.ds(seg0a, CAP)], codes_v)
        pltpu.sync_copy(pm_hbm.at[pl.ds(seg0a, CAP)], perm_v)

        lo = seg0 - seg0a
        hi = wend - seg0a
        for t in range(CAP // LANES):
            i16 = t * LANES + d16
            iv = perm_v[pl.ds(t * LANES, LANES)]
            keep = (i16 >= lo) & (i16 < hi)
            perm_v[pl.ds(t * LANES, LANES)] = jnp.where(keep, iv, batch)

        def issue(j, tile, sem):
            vb = jnp.minimum(wid * bpw + j, max_vb)
            for t in range(DSPLIT):
                rows = dim // DSPLIT
                pltpu.async_copy(
                    wt_hbm.at[pl.ds(t * rows, rows),
                              pl.ds(vb * BLK, BLK)],
                    tile.at[pl.ds(t * rows, rows)], sem)

        for t in range(NBUF):
            issue(t, tiles[t], sems[t])

        def ring_body(i, k):
            for t in range(NBUF):
                j = NBUF * i + t
                tile = tiles[t]
                sem = sems[t]
                pltpu.make_async_copy(
                    wt_hbm.at[:, pl.ds(0, BLK)], tile, sem).wait()
                b_here = wid * bpw + j

                def wcond(k2):
                    k_off = jnp.minimum(k2 - seg0a, CAP - LANES)
                    c = codes_v[pl.ds(k_off, LANES)][0]
                    return (k2 < wend) & ((c >> 7) == b_here)

                def wbody(k2):
                    k_off = jnp.minimum(k2 - seg0a, CAP - LANES)
                    c = codes_v[pl.ds(k_off, LANES)][0]
                    lane = c & (BLK - 1)
                    pos = jnp.minimum(k2 - seg0a, CAP - 1)
                    cols = jnp.full((LANES,), lane, jnp.int32)
                    for m in range(dim // LANES):
                        col = plsc.load_gather(tile, [m * LANES + d16, cols])
                        rows_v[pos, pl.ds(m * LANES, LANES)] = col
                    return k2 + 1

                k = lax.while_loop(wcond, wbody, k)
                issue(j + NBUF, tile, sem)
            return k

        lax.fori_loop(0, n_slots // NBUF, ring_body, seg0)
        for t in range(NBUF):
            pltpu.make_async_copy(
                wt_hbm.at[:, pl.ds(0, BLK)], tiles[t], sems[t]).wait()

        pltpu.async_copy(rows_v, e_hbm.at[perm_v], sem_sc).wait()


def _dot_body(e0_hbm, e1_hbm, c0_hbm, c1_hbm, b0_hbm, b1_hbm, out_hbm,
              e0_v, e1_v, c0_v, c1_v, bb0_v, bb1_v, out_v, sem):
    b_per_w = out_v.shape[0]
    chunk = e0_v.shape[0]
    dim = e0_v.shape[1] // 2
    wid = lax.axis_index("s") * NUM_CORES + lax.axis_index("c")
    base = wid * b_per_w
    d16 = lax.iota(jnp.int32, LANES)

    for c in range(b_per_w // chunk):
        cbase = base + c * chunk
        pltpu.sync_copy(c0_hbm.at[pl.ds(cbase, chunk)], c0_v)
        pltpu.sync_copy(c1_hbm.at[pl.ds(cbase, chunk)], c1_v)
        copies = [
            pltpu.async_copy(e0_hbm.at[pl.ds(cbase, chunk)], e0_v, sem),
            pltpu.async_copy(e1_hbm.at[pl.ds(cbase, chunk)], e1_v, sem),
            pltpu.async_copy(b0_hbm.at[c0_v], bb0_v, sem),
            pltpu.async_copy(b1_hbm.at[c1_v], bb1_v, sem),
        ]
        for cp in copies:
            cp.wait()

        def grp_body(g, carry):
            s = g * LANES
            rows = s + d16
            acc = bb0_v[pl.ds(s, LANES)] + bb1_v[pl.ds(s, LANES)]
            for d in range(dim):
                cols = jnp.full((LANES,), d, jnp.int32)
                acc = acc + plsc.load_gather(e0_v, [rows, cols]) * \
                    plsc.load_gather(e1_v, [rows, cols])
            out_v[pl.ds(c * chunk + s, LANES)] = acc
            return carry

        lax.fori_loop(0, chunk // LANES, grp_body, 0)

    pltpu.sync_copy(out_v, out_hbm.at[pl.ds(base, b_per_w)])


def kernel(x, W0, W1, b0, b1):
    batch = x.shape[0]
    vocab, dim = W0.shape
    n_blocks = (vocab + BLK - 1) // BLK
    bpw = (n_blocks + NUM_WORKERS - 1) // NUM_WORKERS
    b_per_w = batch // NUM_WORKERS

    codes0 = x[:, 0].astype(jnp.int32)
    codes1 = x[:, 1].astype(jnp.int32)

    def field_setup(codes):
        sc, perm = lax.sort_key_val(
            codes, jnp.arange(batch, dtype=jnp.int32))
        qs = (jnp.arange(48, dtype=jnp.int32) * bpw * BLK)
        ws = jnp.searchsorted(sc, qs).astype(jnp.int32)
        scp = jnp.pad(sc, (0, CAP))
        pmp = jnp.pad(perm, (0, CAP), constant_values=batch)
        return scp, ws, pmp

    sc0p, ws0, pm0 = field_setup(codes0)
    sc1p, ws1, pm1 = field_setup(codes1)

    mesh = plsc.VectorSubcoreMesh(core_axis_name="c", subcore_axis_name="s")
    params = pltpu.CompilerParams(
        needs_layout_passes=False, use_tc_tiling_on_sc=True)

    extract = pl.kernel(
        _extract_body,
        out_type=(jax.ShapeDtypeStruct((batch + 1, 2 * dim), jnp.float32),
                  jax.ShapeDtypeStruct((batch + 1, 2 * dim), jnp.float32)),
        mesh=mesh,
        compiler_params=params,
        scratch_types=[
            pltpu.VMEM((dim, BLK), jnp.float32),
            pltpu.VMEM((dim, BLK), jnp.float32),
            pltpu.VMEM((dim, BLK), jnp.float32),
            pltpu.VMEM((CAP, 2 * dim), jnp.float32),
            pltpu.VMEM((CAP,), jnp.int32),
            pltpu.VMEM((CAP,), jnp.int32),
            pltpu.VMEM((48,), jnp.int32),
            pltpu.SemaphoreType.DMA,
            pltpu.SemaphoreType.DMA,
            pltpu.SemaphoreType.DMA,
            pltpu.SemaphoreType.DMA,
        ],
    )
    e0, e1 = extract(sc0p, sc1p, ws0, ws1, pm0, pm1, W0.T, W1.T)

    dot = pl.kernel(
        _dot_body,
        out_type=jax.ShapeDtypeStruct((batch,), jnp.float32),
        mesh=mesh,
        compiler_params=params,
        scratch_types=[
            pltpu.VMEM((b_per_w // 2, 2 * dim), jnp.float32),
            pltpu.VMEM((b_per_w // 2, 2 * dim), jnp.float32),
            pltpu.VMEM((b_per_w // 2,), jnp.int32),
            pltpu.VMEM((b_per_w // 2,), jnp.int32),
            pltpu.VMEM((b_per_w // 2,), jnp.float32),
            pltpu.VMEM((b_per_w // 2,), jnp.float32),
            pltpu.VMEM((b_per_w,), jnp.float32),
            pltpu.SemaphoreType.DMA,
        ],
    )
    return dot(e0, e1, codes0, codes1, b0.reshape(-1), b1.reshape(-1))
